# baseline (speedup 1.0000x reference)
import functools

import jax
import jax.numpy as jnp
from jax import lax
from jax.experimental import pallas as pl
from jax.experimental.pallas import tpu as pltpu
from jax.experimental.pallas import tpu_sc as plsc

NC, NS, LANES = 2, 16, 16
NW = NC * NS
EPS = 1e-5


def _make_encoder(n_rows, d_model, seq_len, pos_rows, c_rows):
    assert d_model % LANES == 0
    n_per_w = n_rows // NW
    n_chunks = n_per_w // c_rows
    k_idx = c_rows // 128
    d_chunks = d_model // LANES
    mesh = plsc.VectorSubcoreMesh(core_axis_name="c", subcore_axis_name="s")

    @functools.partial(
        pl.kernel,
        out_type=jax.ShapeDtypeStruct((n_rows, d_model), jnp.float32),
        mesh=mesh,
        compiler_params=pltpu.CompilerParams(
            needs_layout_passes=False, use_tc_tiling_on_sc=False),
        scratch_types=[
            pltpu.VMEM((c_rows,), jnp.int32),
            pltpu.VMEM((c_rows, d_model), jnp.float32),
            pltpu.VMEM((pos_rows * d_model,), jnp.float32),
            pltpu.VMEM((d_model,), jnp.float32),
            pltpu.VMEM((d_model,), jnp.float32),
            pltpu.SemaphoreType.DMA,
        ],
    )
    def enc(tok_hbm, table_hbm, pos_hbm, gamma_hbm, beta_hbm, out_hbm,
            idx_v, rows_v, pos_v, gamma_v, beta_v, sem):
        wid = lax.axis_index("s") * NC + lax.axis_index("c")
        perms = [lax.iota(jnp.int32, LANES) ^ k for k in (1, 2, 4, 8)]

        def lane_sum(v):
            for p in perms:
                v = v + v.at[p].get(mode="promise_in_bounds")
            return v

        pltpu.sync_copy(pos_hbm, pos_v)
        pltpu.sync_copy(gamma_hbm, gamma_v)
        pltpu.sync_copy(beta_hbm, beta_v)
        g = [gamma_v[pl.ds(c * LANES, LANES)] for c in range(d_chunks)]
        bt = [beta_v[pl.ds(c * LANES, LANES)] for c in range(d_chunks)]
        row0 = wid * n_per_w

        @pl.loop(0, n_chunks)
        def _chunk(k):
            base = row0 + k * c_rows
            pltpu.sync_copy(tok_hbm.at[pl.ds(base, c_rows)], idx_v)
            copies = [
                pltpu.async_copy(table_hbm.at[idx_v.at[pl.ds(j * 128, 128)]],
                                 rows_v.at[pl.ds(j * 128, 128)], sem)
                for j in range(k_idx)
            ]
            for cp in copies:
                cp.wait()

            @pl.loop(0, c_rows)
            def _row(r):
                pb = lax.rem(base + r, seq_len) * d_model
                x = [rows_v[r, pl.ds(c * LANES, LANES)]
                     + pos_v[pl.ds(pb + c * LANES, LANES)]
                     for c in range(d_chunks)]
                s = x[0]
                q = x[0] * x[0]
                for c in range(1, d_chunks):
                    s = s + x[c]
                    q = q + x[c] * x[c]
                mean = lane_sum(s) * (1.0 / d_model)
                var = lane_sum(q) * (1.0 / d_model) - mean * mean
                vv = var + EPS
                iv = plsc.bitcast(vv, jnp.int32)
                y = plsc.bitcast(jnp.int32(0x5F3759DF)
                                 - lax.shift_right_logical(iv, 1), jnp.float32)
                y = y * (1.5 - 0.5 * vv * y * y)
                y = y * (1.5 - 0.5 * vv * y * y)
                for c in range(d_chunks):
                    rows_v[r, pl.ds(c * LANES, LANES)] = (
                        (x[c] - mean) * y * g[c] + bt[c])

            pltpu.sync_copy(rows_v, out_hbm.at[pl.ds(base, c_rows)])

    return enc


def kernel(tokens, token_table, pos_table, gamma, beta):
    b, l = tokens.shape
    d = token_table.shape[1]
    n = b * l
    tok2 = tokens.astype(jnp.int32).reshape(n)
    posf = pos_table.reshape(-1).astype(jnp.float32)
    enc = _make_encoder(n, d, l, pos_table.shape[0], 640)
    out = enc(tok2, token_table, posf, gamma, beta)
    return out.reshape(b, l, d)


# speedup vs baseline: 1.0143x; 1.0143x over previous
import functools

import jax
import jax.numpy as jnp
from jax import lax
from jax.experimental import pallas as pl
from jax.experimental.pallas import tpu as pltpu
from jax.experimental.pallas import tpu_sc as plsc

NC, NS, LANES = 2, 16, 16
NW = NC * NS
EPS = 1e-5
C_ROWS = 128


def _make_encoder(n_rows, d_model, seq_len, pos_rows):
    assert d_model == 64
    n_per_w = n_rows // NW
    n_chunks = n_per_w // C_ROWS
    assert n_chunks % 2 == 0
    mesh = plsc.VectorSubcoreMesh(core_axis_name="c", subcore_axis_name="s")

    @functools.partial(
        pl.kernel,
        out_type=jax.ShapeDtypeStruct((n_rows // 2, 2 * d_model), jnp.float32),
        mesh=mesh,
        compiler_params=pltpu.CompilerParams(needs_layout_passes=False),
        scratch_types=[
            pltpu.VMEM((n_per_w,), jnp.int32),
            pltpu.VMEM((n_per_w,), jnp.int32),
            pltpu.VMEM((C_ROWS, 2 * d_model), jnp.float32),
            pltpu.VMEM((C_ROWS, 2 * d_model), jnp.float32),
            pltpu.VMEM((C_ROWS // 2, 2 * d_model), jnp.float32),
            pltpu.VMEM((C_ROWS // 2, 2 * d_model), jnp.float32),
            pltpu.VMEM((pos_rows * d_model,), jnp.float32),
            pltpu.VMEM((d_model,), jnp.float32),
            pltpu.VMEM((d_model,), jnp.float32),
            pltpu.SemaphoreType.DMA,
            pltpu.SemaphoreType.DMA,
            pltpu.SemaphoreType.DMA,
            pltpu.SemaphoreType.DMA,
        ],
    )
    def enc(tok_hbm, table_hbm, pos_hbm, gamma_hbm, beta_hbm, out_hbm,
            idx_v, idxh_v, rows_a, rows_b, out_a, out_b, pos_v,
            gamma_v, beta_v, sem_a, sem_b, semw_a, semw_b):
        wid = lax.axis_index("s") * NC + lax.axis_index("c")
        row0 = wid * n_per_w
        pltpu.sync_copy(tok_hbm.at[pl.ds(row0, n_per_w)], idx_v)
        pltpu.sync_copy(pos_hbm, pos_v)
        pltpu.sync_copy(gamma_hbm, gamma_v)
        pltpu.sync_copy(beta_hbm, beta_v)
        g = [gamma_v[pl.ds(c * LANES, LANES)] for c in range(4)]
        bt = [beta_v[pl.ds(c * LANES, LANES)] for c in range(4)]
        perms = [lax.iota(jnp.int32, LANES) ^ k for k in (1, 2, 4, 8)]

        @pl.loop(0, n_per_w // LANES)
        def _half(i):
            idxh_v[pl.ds(i * LANES, LANES)] = lax.shift_right_logical(
                idx_v[pl.ds(i * LANES, LANES)], 1)

        def lane_sum(v):
            for p in perms:
                v = v + v.at[p].get(mode="promise_in_bounds")
            return v

        rows = (rows_a, rows_b)
        outs = (out_a, out_b)
        sems = (sem_a, sem_b)
        semws = (semw_a, semw_b)

        def start_gather(k, buf, sem):
            pltpu.async_copy(
                table_hbm.at[idxh_v.at[pl.ds(k * C_ROWS, C_ROWS)]], buf, sem)

        def process(k, b):
            pltpu.make_async_copy(
                table_hbm.at[idxh_v.at[pl.ds(k * C_ROWS, C_ROWS)]],
                rows[b], sems[b]).wait()
            rv, ov = rows[b], outs[b]

            @pl.when(k >= 2)
            def _():
                pltpu.make_async_copy(
                    ov, out_hbm.at[pl.ds(
                        pl.multiple_of((row0 + (k - 2) * C_ROWS) // 2, 8),
                        C_ROWS // 2)], semws[b]).wait()

            @pl.loop(0, C_ROWS, step=LANES)
            def _rowgrp(rg):
                tvec = idx_v[pl.ds(k * C_ROWS + rg, LANES)]
                for j in range(LANES):
                    r = rg + j
                    hoff = (tvec[j] & 1) * d_model
                    pb = lax.rem(row0 + k * C_ROWS + r, seq_len) * d_model
                    x = [rv[r, pl.ds(hoff + c * LANES, LANES)]
                         + pos_v[pl.ds(pb + c * LANES, LANES)]
                         for c in range(4)]
                    s = (x[0] + x[1]) + (x[2] + x[3])
                    q = (x[0] * x[0] + x[1] * x[1]
                         + x[2] * x[2] + x[3] * x[3])
                    mean = lane_sum(s) * (1.0 / d_model)
                    vv = lane_sum(q) * (1.0 / d_model) - mean * mean + EPS
                    iv = plsc.bitcast(vv, jnp.int32)
                    y = plsc.bitcast(
                        jnp.int32(0x5F3759DF)
                        - lax.shift_right_logical(iv, 1), jnp.float32)
                    y = y * (1.5 - 0.5 * vv * y * y)
                    y = y * (1.5 - 0.5 * vv * y * y)
                    orow = lax.shift_right_logical(r, 1)
                    ocol = (r & 1) * d_model
                    for c in range(4):
                        ov[orow, pl.ds(ocol + c * LANES, LANES)] = (
                            (x[c] - mean) * y * g[c] + bt[c])

            pltpu.async_copy(
                ov, out_hbm.at[pl.ds(
                    pl.multiple_of((row0 + k * C_ROWS) // 2, 8),
                    C_ROWS // 2)], semws[b])

            @pl.when(k + 2 < n_chunks)
            def _():
                start_gather(k + 2, rv, sems[b])

        start_gather(0, rows[0], sems[0])
        start_gather(1, rows[1], sems[1])

        @pl.loop(0, n_chunks, step=2)
        def _pair(k):
            process(k, 0)
            process(k + 1, 1)

        for b, k in ((0, n_chunks - 2), (1, n_chunks - 1)):
            pltpu.make_async_copy(
                outs[b], out_hbm.at[pl.ds(
                    pl.multiple_of((row0 + k * C_ROWS) // 2, 8),
                    C_ROWS // 2)], semws[b]).wait()

    return enc


def kernel(tokens, token_table, pos_table, gamma, beta):
    b, l = tokens.shape
    v, d = token_table.shape
    n = b * l
    tok_flat = tokens.astype(jnp.int32).reshape(n)
    table2 = token_table.reshape(v // 2, 2 * d)
    posf = pos_table.reshape(-1).astype(jnp.float32)
    enc = _make_encoder(n, d, l, pos_table.shape[0])
    out2 = enc(tok_flat, table2, posf, gamma, beta)
    return out2.reshape(b, l, d)
